# baseline (speedup 1.0000x reference)
import functools

import jax
import jax.numpy as jnp
from jax import lax
from jax.experimental import pallas as pl
from jax.experimental.pallas import tpu as pltpu
from jax.experimental.pallas import tpu_sc as plsc

_BT = 256
_M = 128


def _silu(x):
    return x * jax.nn.sigmoid(x)


def _bf16_as_i32(a):
    n = a.shape[0]
    return lax.bitcast_convert_type(a.reshape(n, -1, 2), jnp.int32)


def _i32_as_bf16(a):
    n = a.shape[0]
    return lax.bitcast_convert_type(a, jnp.bfloat16).reshape(n, -1)


def _router_body(x_ref, wr_ref, logits_ref, ti_ref, tw_ref):
    xb = x_ref[...]
    logits = jnp.dot(xb, wr_ref[...], preferred_element_type=jnp.float32)
    logits_ref[...] = logits
    probs = jax.nn.softmax(logits, axis=-1)
    eidx = lax.broadcasted_iota(jnp.int32, probs.shape, 1)
    big = jnp.int32(probs.shape[1])
    m1 = jnp.max(probs, axis=-1, keepdims=True)
    i1 = jnp.min(jnp.where(probs == m1, eidx, big), axis=-1, keepdims=True)
    masked = jnp.where(eidx == i1, -jnp.inf, probs)
    m2 = jnp.max(masked, axis=-1, keepdims=True)
    i2 = jnp.min(jnp.where(masked == m2, eidx, big), axis=-1, keepdims=True)
    ssum = m1 + m2
    ti_ref[...] = jnp.concatenate([i1, i2], axis=1)
    tw_ref[...] = jnp.concatenate([m1 / ssum, m2 / ssum], axis=1)


def _run_router(x, Wr):
    t, h = x.shape
    e_num = Wr.shape[1]
    bt = 512
    return pl.pallas_call(
        _router_body,
        grid=(t // bt,),
        in_specs=[
            pl.BlockSpec((bt, h), lambda i: (i, 0)),
            pl.BlockSpec((h, e_num), lambda i: (0, 0)),
        ],
        out_specs=[
            pl.BlockSpec((bt, e_num), lambda i: (i, 0)),
            pl.BlockSpec((bt, 2), lambda i: (i, 0)),
            pl.BlockSpec((bt, 2), lambda i: (i, 0)),
        ],
        out_shape=[
            jax.ShapeDtypeStruct((t, e_num), jnp.float32),
            jax.ShapeDtypeStruct((t, 2), jnp.int32),
            jax.ShapeDtypeStruct((t, 2), jnp.float32),
        ],
        compiler_params=pltpu.CompilerParams(
            dimension_semantics=("parallel",),
        ),
    )(x, Wr)


def _shared_body(x_ref, sg_ref, su_ref, sd_ref, shared_ref):
    xh = x_ref[...].astype(jnp.bfloat16)
    g = jnp.dot(xh, sg_ref[...], preferred_element_type=jnp.float32)
    u = jnp.dot(xh, su_ref[...], preferred_element_type=jnp.float32)
    shared_ref[...] = jnp.dot((_silu(g) * u).astype(jnp.bfloat16), sd_ref[...],
                              preferred_element_type=jnp.float32)


def _run_shared(x, sgh, suh, sdh):
    t, h = x.shape
    fs = sgh.shape[1]
    return pl.pallas_call(
        _shared_body,
        grid=(t // _BT,),
        in_specs=[
            pl.BlockSpec((_BT, h), lambda i: (i, 0)),
            pl.BlockSpec((h, fs), lambda i: (0, 0)),
            pl.BlockSpec((h, fs), lambda i: (0, 0)),
            pl.BlockSpec((fs, h), lambda i: (0, 0)),
        ],
        out_specs=pl.BlockSpec((_BT, h), lambda i: (i, 0)),
        out_shape=jax.ShapeDtypeStruct((t, h), jnp.float32),
        compiler_params=pltpu.CompilerParams(
            dimension_semantics=("parallel",),
        ),
    )(x, sgh, suh, sdh)


def _dispatch_body(e_num, tk, nt, ti_ref, stok_ref, te_ref, pos_ref,
                   eid_v, stok_v, pos_v, merge_v, macc_v, zero_v, te_v,
                   pos_sh):
    cid = lax.axis_index("c")
    sid = lax.axis_index("s")
    nchunk = tk // 16
    iota16 = lax.iota(jnp.int32, 16)
    zeros16 = jnp.zeros((16,), jnp.int32)

    @pl.when((cid == 0) & (sid < e_num))
    def _expert_work():
        pltpu.sync_copy(ti_ref, eid_v)

        def zero_pos(i, _):
            pos_v[pl.ds(i * 16, 16)] = zeros16
            return 0
        lax.fori_loop(0, nchunk, zero_pos, 0)

        def zero_stok(i, _):
            stok_v[pl.ds(i * 16, 16)] = zeros16
            return 0
        lax.fori_loop(0, nchunk + 8, zero_stok, 0)
        for zz in range(_M // 16):
            zero_v[pl.ds(zz * 16, 16)] = zeros16

        def cnt_body(i, acc):
            chunk = eid_v[pl.ds(i * 16, 16)]
            return tuple(acc[e] + (chunk == e).astype(jnp.int32)
                         for e in range(e_num))
        acc0 = tuple(zeros16 for _ in range(e_num))
        acc = lax.fori_loop(0, nchunk, cnt_body, acc0)
        cnt = [jnp.sum(a) for a in acc]
        ntile = [(c + (_M - 1)) // _M for c in cnt]
        base = []
        run = jnp.int32(0)
        for e in range(e_num):
            base.append(run)
            run = run + ntile[e] * _M
        used = run // _M
        my_base = jnp.int32(0)
        my_ntile = jnp.int32(0)
        for e in range(e_num):
            my_base = jnp.where(sid == e, base[e], my_base)
            my_ntile = jnp.where(sid == e, ntile[e], my_ntile)

        def scan_body(i, off):
            chunk = eid_v[pl.ds(i * 16, 16)]
            m = chunk == sid
            jvec = i * 16 + iota16
            tok = jvec >> 1
            cum = plsc.cumsum(m.astype(jnp.int32))
            plsc.store_compressed(stok_v.at[pl.ds(off, 16)], tok, mask=m)
            posvals = my_base + off + cum - 1
            plsc.store_scatter(pos_v, [jvec], posvals, mask=m)
            return off + jnp.max(cum)
        lax.fori_loop(0, nchunk, scan_body, jnp.int32(0))

        def dma_body(i, _):
            dst = pl.multiple_of(my_base + i * _M, _M)
            pltpu.sync_copy(stok_v.at[pl.ds(i * _M, _M)],
                            stok_ref.at[pl.ds(dst, _M)])
            return 0
        lax.fori_loop(0, my_ntile, dma_body, 0)

        pltpu.sync_copy(pos_v, pos_sh.at[sid])

        @pl.when(sid == 0)
        def _meta():
            for m_t in range(nt):
                @pl.when(m_t >= used)
                def _zt():
                    pltpu.sync_copy(zero_v, stok_ref.at[pl.ds(m_t * _M, _M)])
            tbs = [b // _M for b in base]
            for ch in range((nt + 15) // 16):
                mvec = ch * 16 + iota16
                te = jnp.zeros((16,), jnp.int32)
                for e in range(e_num):
                    inseg = (mvec >= tbs[e]) & (mvec < tbs[e] + ntile[e])
                    te = te + e * inseg.astype(jnp.int32)
                te_v[pl.ds(ch * 16, 16)] = te
            pltpu.sync_copy(te_v, te_ref)

    plsc.subcore_barrier()

    @pl.when(cid == 0)
    def _merge():
        seg = tk // 16
        for e in range(e_num):
            pltpu.sync_copy(pos_sh.at[e, pl.ds(sid * seg, seg)],
                            merge_v.at[e])

        def add_body(i, _):
            sl = pl.ds(i * 16, 16)
            s = merge_v[0, sl]
            for e in range(1, e_num):
                s = s + merge_v[e, sl]
            macc_v[sl] = s
            return 0
        lax.fori_loop(0, seg // 16, add_body, 0)
        dst = pl.multiple_of(sid * seg, seg)
        pltpu.sync_copy(macc_v, pos_ref.at[pl.ds(dst, seg)])


def _run_dispatch(ti_flat, e_num, tk, nt):
    mesh = plsc.VectorSubcoreMesh(core_axis_name="c", subcore_axis_name="s")
    seg = tk // 16
    body = functools.partial(_dispatch_body, e_num, tk, nt)
    f = pl.kernel(
        body,
        out_type=[
            jax.ShapeDtypeStruct((nt * _M,), jnp.int32),
            jax.ShapeDtypeStruct((((nt + 15) // 16) * 16,), jnp.int32),
            jax.ShapeDtypeStruct((tk,), jnp.int32),
        ],
        mesh=mesh,
        compiler_params=pltpu.CompilerParams(needs_layout_passes=False),
        scratch_types=[
            pltpu.VMEM((tk,), jnp.int32),
            pltpu.VMEM((tk + 128,), jnp.int32),
            pltpu.VMEM((tk,), jnp.int32),
            pltpu.VMEM((e_num, seg), jnp.int32),
            pltpu.VMEM((seg,), jnp.int32),
            pltpu.VMEM((_M,), jnp.int32),
            pltpu.VMEM((((nt + 15) // 16) * 16,), jnp.int32),
            pltpu.VMEM_SHARED((e_num, tk), jnp.int32),
        ],
    )
    return f(ti_flat)


def _gather_body(rows_per_w, h, stok_ref, x_ref, xs_ref, idx_v, rows_v, sem):
    wid = lax.axis_index("s") * 2 + lax.axis_index("c")
    half = rows_per_w // 2
    s0 = pl.multiple_of(wid * rows_per_w, 8)
    pltpu.sync_copy(stok_ref.at[pl.ds(s0, rows_per_w)], idx_v)
    d0 = pltpu.async_copy(x_ref.at[idx_v.at[pl.ds(0, half)]],
                          rows_v.at[pl.ds(0, half)], sem)
    d1 = pltpu.async_copy(x_ref.at[idx_v.at[pl.ds(half, half)]],
                          rows_v.at[pl.ds(half, half)], sem)
    d0.wait()
    d1.wait()
    pltpu.sync_copy(rows_v, xs_ref.at[pl.ds(s0, rows_per_w)])


def _run_gather(stok, xbf, npad):
    t, h = xbf.shape
    rows_per_w = npad // 32
    mesh = plsc.VectorSubcoreMesh(core_axis_name="c", subcore_axis_name="s")
    body = functools.partial(_gather_body, rows_per_w, h)
    f = pl.kernel(
        body,
        out_type=jax.ShapeDtypeStruct((npad, h), jnp.int32),
        mesh=mesh,
        compiler_params=pltpu.CompilerParams(needs_layout_passes=False),
        scratch_types=[
            pltpu.VMEM((rows_per_w,), jnp.int32),
            pltpu.VMEM((rows_per_w, h), jnp.int32),
            pltpu.SemaphoreType.DMA,
        ],
    )
    return f(stok, xbf)


def _group_ffn_body(te_ref, xs_ref, wg_ref, wu_ref, wd_ref, y_ref):
    xh = xs_ref[...]
    g = jnp.dot(xh, wg_ref[0], preferred_element_type=jnp.float32)
    u = jnp.dot(xh, wu_ref[0], preferred_element_type=jnp.float32)
    y_ref[...] = jnp.dot((_silu(g) * u).astype(jnp.bfloat16), wd_ref[0],
                         preferred_element_type=jnp.float32
                         ).astype(jnp.bfloat16)


def _run_group_ffn(te, xs, Wgh, Wuh, Wdh, nt):
    npad, h = xs.shape
    f = Wgh.shape[2]
    grid_spec = pltpu.PrefetchScalarGridSpec(
        num_scalar_prefetch=1,
        grid=(nt,),
        in_specs=[
            pl.BlockSpec((_M, h), lambda i, te_r: (i, 0)),
            pl.BlockSpec((1, h, f), lambda i, te_r: (te_r[i], 0, 0)),
            pl.BlockSpec((1, h, f), lambda i, te_r: (te_r[i], 0, 0)),
            pl.BlockSpec((1, f, h), lambda i, te_r: (te_r[i], 0, 0)),
        ],
        out_specs=pl.BlockSpec((_M, h), lambda i, te_r: (i, 0)),
    )
    return pl.pallas_call(
        _group_ffn_body,
        grid_spec=grid_spec,
        out_shape=jax.ShapeDtypeStruct((npad, h), jnp.bfloat16),
        compiler_params=pltpu.CompilerParams(
            dimension_semantics=("arbitrary",),
        ),
    )(te, xs, Wgh, Wuh, Wdh)


def _ygather_body(rows_per_w, h, pos_ref, y_ref, yg_ref, pos_v, rows_v, sem):
    wid = lax.axis_index("s") * 2 + lax.axis_index("c")
    s0 = pl.multiple_of(wid * rows_per_w, 8)
    pltpu.sync_copy(pos_ref.at[pl.ds(s0, rows_per_w)], pos_v)
    pltpu.async_copy(y_ref.at[pos_v], rows_v, sem).wait()
    pltpu.sync_copy(rows_v, yg_ref.at[pl.ds(s0, rows_per_w)])


def _run_ygather(pos, y):
    tk = pos.shape[0]
    h = y.shape[1]
    rows_per_w = tk // 32
    mesh = plsc.VectorSubcoreMesh(core_axis_name="c", subcore_axis_name="s")
    body = functools.partial(_ygather_body, rows_per_w, h)
    f = pl.kernel(
        body,
        out_type=jax.ShapeDtypeStruct((tk, h), jnp.int32),
        mesh=mesh,
        compiler_params=pltpu.CompilerParams(needs_layout_passes=False),
        scratch_types=[
            pltpu.VMEM((rows_per_w,), jnp.int32),
            pltpu.VMEM((rows_per_w, h), jnp.int32),
            pltpu.SemaphoreType.DMA,
        ],
    )
    return f(pos, y)


def _combine_body(sh_ref, w_ref, yg_ref, out_ref):
    h = sh_ref.shape[1]
    w0 = w_ref[:, 0:1]
    w1 = w_ref[:, 1:2]
    y0 = yg_ref[:, :h].astype(jnp.float32)
    y1 = yg_ref[:, h:].astype(jnp.float32)
    out_ref[...] = sh_ref[...] + w0 * y0 + w1 * y1


def _run_combine(shared, tw, ygr):
    t, h = shared.shape
    return pl.pallas_call(
        _combine_body,
        grid=(t // _BT,),
        in_specs=[
            pl.BlockSpec((_BT, h), lambda i: (i, 0)),
            pl.BlockSpec((_BT, 2), lambda i: (i, 0)),
            pl.BlockSpec((_BT, 2 * h), lambda i: (i, 0)),
        ],
        out_specs=pl.BlockSpec((_BT, h), lambda i: (i, 0)),
        out_shape=jax.ShapeDtypeStruct((t, h), jnp.float32),
        compiler_params=pltpu.CompilerParams(
            dimension_semantics=("parallel",),
        ),
    )(shared, tw, ygr)


def kernel(hidden_states, Wr, sg, su, sd, Wg, Wu, Wd):
    b, s, h = hidden_states.shape
    t = b * s
    x = hidden_states.reshape(t, h)
    e_num = Wr.shape[1]
    k = 2
    tk = t * k
    nt = tk // _M + e_num
    npad = nt * _M

    sgh = sg.astype(jnp.bfloat16)
    suh = su.astype(jnp.bfloat16)
    sdh = sd.astype(jnp.bfloat16)
    Wgh = Wg.astype(jnp.bfloat16)
    Wuh = Wu.astype(jnp.bfloat16)
    Wdh = Wd.astype(jnp.bfloat16)
    xbf = x.astype(jnp.bfloat16)

    logits, ti, tw = _run_router(x, Wr)
    ti_flat = ti.reshape(tk)

    stok, te, pos = _run_dispatch(ti_flat, e_num, tk, nt)
    xs32 = _run_gather(stok, _bf16_as_i32(xbf), npad)
    y = _run_group_ffn(te, _i32_as_bf16(xs32), Wgh, Wuh, Wdh, nt)
    shared = _run_shared(x, sgh, suh, sdh)
    yg32 = _run_ygather(pos, _bf16_as_i32(y))
    out = _run_combine(shared, tw, _i32_as_bf16(yg32).reshape(t, 2 * h))
    return out.reshape(b, s, h), logits


# speedup vs baseline: 9.9621x; 9.9621x over previous
import functools

import jax
import jax.numpy as jnp
from jax import lax
from jax.experimental import pallas as pl
from jax.experimental.pallas import tpu as pltpu
from jax.experimental.pallas import tpu_sc as plsc

_BT = 256
_M = 128


def _silu(x):
    return x * jax.nn.sigmoid(x)


def _bf16_as_i32(a):
    n = a.shape[0]
    return lax.bitcast_convert_type(a.reshape(n, -1, 2), jnp.int32)


def _i32_as_bf16(a):
    n = a.shape[0]
    return lax.bitcast_convert_type(a, jnp.bfloat16).reshape(n, -1)


def _router_body(x_ref, wr_ref, logits_ref, ti_ref, tw_ref):
    xb = x_ref[...]
    logits = jnp.dot(xb, wr_ref[...], preferred_element_type=jnp.float32)
    logits_ref[...] = logits
    probs = jax.nn.softmax(logits, axis=-1)
    eidx = lax.broadcasted_iota(jnp.int32, probs.shape, 1)
    big = jnp.int32(probs.shape[1])
    m1 = jnp.max(probs, axis=-1, keepdims=True)
    i1 = jnp.min(jnp.where(probs == m1, eidx, big), axis=-1, keepdims=True)
    masked = jnp.where(eidx == i1, -jnp.inf, probs)
    m2 = jnp.max(masked, axis=-1, keepdims=True)
    i2 = jnp.min(jnp.where(masked == m2, eidx, big), axis=-1, keepdims=True)
    ssum = m1 + m2
    ti_ref[...] = jnp.concatenate([i1, i2], axis=1)
    tw_ref[...] = jnp.concatenate([m1 / ssum, m2 / ssum], axis=1)


def _run_router(x, Wr):
    t, h = x.shape
    e_num = Wr.shape[1]
    bt = 512
    return pl.pallas_call(
        _router_body,
        grid=(t // bt,),
        in_specs=[
            pl.BlockSpec((bt, h), lambda i: (i, 0)),
            pl.BlockSpec((h, e_num), lambda i: (0, 0)),
        ],
        out_specs=[
            pl.BlockSpec((bt, e_num), lambda i: (i, 0)),
            pl.BlockSpec((bt, 2), lambda i: (i, 0)),
            pl.BlockSpec((bt, 2), lambda i: (i, 0)),
        ],
        out_shape=[
            jax.ShapeDtypeStruct((t, e_num), jnp.float32),
            jax.ShapeDtypeStruct((t, 2), jnp.int32),
            jax.ShapeDtypeStruct((t, 2), jnp.float32),
        ],
        compiler_params=pltpu.CompilerParams(
            dimension_semantics=("parallel",),
        ),
    )(x, Wr)


def _shared_body(x_ref, sg_ref, su_ref, sd_ref, shared_ref):
    xh = x_ref[...].astype(jnp.bfloat16)
    g = jnp.dot(xh, sg_ref[...], preferred_element_type=jnp.float32)
    u = jnp.dot(xh, su_ref[...], preferred_element_type=jnp.float32)
    shared_ref[...] = jnp.dot((_silu(g) * u).astype(jnp.bfloat16), sd_ref[...],
                              preferred_element_type=jnp.float32)


def _run_shared(x, sgh, suh, sdh):
    t, h = x.shape
    fs = sgh.shape[1]
    return pl.pallas_call(
        _shared_body,
        grid=(t // _BT,),
        in_specs=[
            pl.BlockSpec((_BT, h), lambda i: (i, 0)),
            pl.BlockSpec((h, fs), lambda i: (0, 0)),
            pl.BlockSpec((h, fs), lambda i: (0, 0)),
            pl.BlockSpec((fs, h), lambda i: (0, 0)),
        ],
        out_specs=pl.BlockSpec((_BT, h), lambda i: (i, 0)),
        out_shape=jax.ShapeDtypeStruct((t, h), jnp.float32),
        compiler_params=pltpu.CompilerParams(
            dimension_semantics=("parallel",),
        ),
    )(x, sgh, suh, sdh)


def _dispatch_body(e_num, tk, nt, ti_ref, stok_ref, te_ref, pos_ref,
                   eid_v, stok_v, pos_v, merge_v, macc_v, zero_v, te_v,
                   pos_sh):
    cid = lax.axis_index("c")
    sid = lax.axis_index("s")
    nchunk = tk // 16
    iota16 = lax.iota(jnp.int32, 16)
    zeros16 = jnp.zeros((16,), jnp.int32)

    @pl.when((cid == 0) & (sid < e_num))
    def _expert_work():
        pltpu.sync_copy(ti_ref, eid_v)

        def zero_pos(i, _):
            pos_v[pl.ds(i * 16, 16)] = zeros16
            return 0
        lax.fori_loop(0, nchunk, zero_pos, 0)

        def zero_stok(i, _):
            stok_v[pl.ds(i * 16, 16)] = zeros16
            return 0
        lax.fori_loop(0, nchunk + 8, zero_stok, 0)
        for zz in range(_M // 16):
            zero_v[pl.ds(zz * 16, 16)] = zeros16

        def cnt_body(i, acc):
            chunk = eid_v[pl.ds(i * 16, 16)]
            return tuple(acc[e] + (chunk == e).astype(jnp.int32)
                         for e in range(e_num))
        acc0 = tuple(zeros16 for _ in range(e_num))
        acc = lax.fori_loop(0, nchunk, cnt_body, acc0)
        cnt = [jnp.sum(a) for a in acc]
        ntile = [(c + (_M - 1)) // _M for c in cnt]
        base = []
        run = jnp.int32(0)
        for e in range(e_num):
            base.append(run)
            run = run + ntile[e] * _M
        used = run // _M
        my_base = jnp.int32(0)
        my_ntile = jnp.int32(0)
        for e in range(e_num):
            my_base = jnp.where(sid == e, base[e], my_base)
            my_ntile = jnp.where(sid == e, ntile[e], my_ntile)

        def scan_body(i, off):
            chunk = eid_v[pl.ds(i * 16, 16)]
            m = chunk == sid
            jvec = i * 16 + iota16
            tok = jvec >> 1
            cum = plsc.cumsum(m.astype(jnp.int32))
            plsc.store_compressed(stok_v.at[pl.ds(off, 16)], tok, mask=m)
            posvals = my_base + off + cum - 1
            plsc.store_scatter(pos_v, [jvec], posvals, mask=m)
            return off + jnp.max(cum)
        lax.fori_loop(0, nchunk, scan_body, jnp.int32(0))

        def dma_body(i, _):
            dst = pl.multiple_of(my_base + i * _M, _M)
            pltpu.sync_copy(stok_v.at[pl.ds(i * _M, _M)],
                            stok_ref.at[pl.ds(dst, _M)])
            return 0
        lax.fori_loop(0, my_ntile, dma_body, 0)

        pltpu.sync_copy(pos_v, pos_sh.at[sid])

        @pl.when(sid == 0)
        def _meta():
            for m_t in range(nt):
                @pl.when(m_t >= used)
                def _zt():
                    pltpu.sync_copy(zero_v, stok_ref.at[pl.ds(m_t * _M, _M)])
            tbs = [b // _M for b in base]
            for ch in range((nt + 15) // 16):
                mvec = ch * 16 + iota16
                te = jnp.zeros((16,), jnp.int32)
                for e in range(e_num):
                    inseg = (mvec >= tbs[e]) & (mvec < tbs[e] + ntile[e])
                    te = te + e * inseg.astype(jnp.int32)
                te_v[pl.ds(ch * 16, 16)] = te
            pltpu.sync_copy(te_v, te_ref)

    plsc.subcore_barrier()

    @pl.when(cid == 0)
    def _merge():
        seg = tk // 16
        for e in range(e_num):
            pltpu.sync_copy(pos_sh.at[e, pl.ds(sid * seg, seg)],
                            merge_v.at[e])

        def add_body(i, _):
            sl = pl.ds(i * 16, 16)
            s = merge_v[0, sl]
            for e in range(1, e_num):
                s = s + merge_v[e, sl]
            macc_v[sl] = s
            return 0
        lax.fori_loop(0, seg // 16, add_body, 0)
        dst = pl.multiple_of(sid * seg, seg)
        pltpu.sync_copy(macc_v, pos_ref.at[pl.ds(dst, seg)])


def _run_dispatch(ti_flat, e_num, tk, nt):
    mesh = plsc.VectorSubcoreMesh(core_axis_name="c", subcore_axis_name="s")
    seg = tk // 16
    body = functools.partial(_dispatch_body, e_num, tk, nt)
    f = pl.kernel(
        body,
        out_type=[
            jax.ShapeDtypeStruct((nt * _M,), jnp.int32),
            jax.ShapeDtypeStruct((((nt + 15) // 16) * 16,), jnp.int32),
            jax.ShapeDtypeStruct((tk,), jnp.int32),
        ],
        mesh=mesh,
        compiler_params=pltpu.CompilerParams(needs_layout_passes=False),
        scratch_types=[
            pltpu.VMEM((tk,), jnp.int32),
            pltpu.VMEM((tk + 128,), jnp.int32),
            pltpu.VMEM((tk,), jnp.int32),
            pltpu.VMEM((e_num, seg), jnp.int32),
            pltpu.VMEM((seg,), jnp.int32),
            pltpu.VMEM((_M,), jnp.int32),
            pltpu.VMEM((((nt + 15) // 16) * 16,), jnp.int32),
            pltpu.VMEM_SHARED((e_num, tk), jnp.int32),
        ],
    )
    return f(ti_flat)


def _gather_body(rows_per_w, h, src_ref, idx_hbm_ref, out_ref,
                 idx_v, buf0, buf1, sem0, sem1):
    wid = lax.axis_index("s") * 2 + lax.axis_index("c")
    s0 = pl.multiple_of(wid * rows_per_w, 8)
    ch = rows_per_w // 4
    pltpu.sync_copy(idx_hbm_ref.at[pl.ds(s0, rows_per_w)], idx_v)
    bufs = [buf0, buf1]
    sems = [sem0, sem1]
    descs = [None] * 4
    descs[0] = pltpu.async_copy(src_ref.at[idx_v.at[pl.ds(0, ch)]],
                                bufs[0], sems[0])
    for c in range(4):
        if c < 3:
            descs[c + 1] = pltpu.async_copy(
                src_ref.at[idx_v.at[pl.ds((c + 1) * ch, ch)]],
                bufs[(c + 1) % 2], sems[(c + 1) % 2])
        descs[c].wait()
        pltpu.sync_copy(bufs[c % 2],
                        out_ref.at[pl.ds(pl.multiple_of(s0 + c * ch, 8), ch)])


def _run_gather(stok, x, npad):
    t, h = x.shape
    rows_per_w = npad // 32
    mesh = plsc.VectorSubcoreMesh(core_axis_name="c", subcore_axis_name="s")
    body = functools.partial(_gather_body, rows_per_w, h)
    f = pl.kernel(
        body,
        out_type=jax.ShapeDtypeStruct((npad, h), jnp.float32),
        mesh=mesh,
        compiler_params=pltpu.CompilerParams(needs_layout_passes=False),
        scratch_types=[
            pltpu.VMEM((rows_per_w,), jnp.int32),
            pltpu.VMEM((rows_per_w // 4, h), jnp.float32),
            pltpu.VMEM((rows_per_w // 4, h), jnp.float32),
            pltpu.SemaphoreType.DMA,
            pltpu.SemaphoreType.DMA,
        ],
    )
    return f(x, stok)


def _group_ffn_body(te_ref, xs_ref, wg_ref, wu_ref, wd_ref, y_ref):
    xh = xs_ref[...].astype(jnp.bfloat16)
    g = jnp.dot(xh, wg_ref[0], preferred_element_type=jnp.float32)
    u = jnp.dot(xh, wu_ref[0], preferred_element_type=jnp.float32)
    y_ref[...] = jnp.dot((_silu(g) * u).astype(jnp.bfloat16), wd_ref[0],
                         preferred_element_type=jnp.float32)


def _run_group_ffn(te, xs, Wgh, Wuh, Wdh, nt):
    npad, h = xs.shape
    f = Wgh.shape[2]
    grid_spec = pltpu.PrefetchScalarGridSpec(
        num_scalar_prefetch=1,
        grid=(nt,),
        in_specs=[
            pl.BlockSpec((_M, h), lambda i, te_r: (i, 0)),
            pl.BlockSpec((1, h, f), lambda i, te_r: (te_r[i], 0, 0)),
            pl.BlockSpec((1, h, f), lambda i, te_r: (te_r[i], 0, 0)),
            pl.BlockSpec((1, f, h), lambda i, te_r: (te_r[i], 0, 0)),
        ],
        out_specs=pl.BlockSpec((_M, h), lambda i, te_r: (i, 0)),
    )
    return pl.pallas_call(
        _group_ffn_body,
        grid_spec=grid_spec,
        out_shape=jax.ShapeDtypeStruct((npad, h), jnp.float32),
        compiler_params=pltpu.CompilerParams(
            dimension_semantics=("arbitrary",),
        ),
    )(te, xs, Wgh, Wuh, Wdh)


def _run_ygather(pos, y):
    tk = pos.shape[0]
    h = y.shape[1]
    rows_per_w = tk // 32
    mesh = plsc.VectorSubcoreMesh(core_axis_name="c", subcore_axis_name="s")
    body = functools.partial(_gather_body, rows_per_w, h)
    f = pl.kernel(
        body,
        out_type=jax.ShapeDtypeStruct((tk, h), jnp.float32),
        mesh=mesh,
        compiler_params=pltpu.CompilerParams(needs_layout_passes=False),
        scratch_types=[
            pltpu.VMEM((rows_per_w,), jnp.int32),
            pltpu.VMEM((rows_per_w // 4, h), jnp.float32),
            pltpu.VMEM((rows_per_w // 4, h), jnp.float32),
            pltpu.SemaphoreType.DMA,
            pltpu.SemaphoreType.DMA,
        ],
    )
    return f(y, pos)


def _combine_body(sh_ref, w_ref, yg_ref, out_ref):
    h = sh_ref.shape[1]
    w0 = w_ref[:, 0:1]
    w1 = w_ref[:, 1:2]
    y0 = yg_ref[:, :h]
    y1 = yg_ref[:, h:]
    out_ref[...] = sh_ref[...] + w0 * y0 + w1 * y1


def _run_combine(shared, tw, ygr):
    t, h = shared.shape
    return pl.pallas_call(
        _combine_body,
        grid=(t // _BT,),
        in_specs=[
            pl.BlockSpec((_BT, h), lambda i: (i, 0)),
            pl.BlockSpec((_BT, 2), lambda i: (i, 0)),
            pl.BlockSpec((_BT, 2 * h), lambda i: (i, 0)),
        ],
        out_specs=pl.BlockSpec((_BT, h), lambda i: (i, 0)),
        out_shape=jax.ShapeDtypeStruct((t, h), jnp.float32),
        compiler_params=pltpu.CompilerParams(
            dimension_semantics=("parallel",),
        ),
    )(shared, tw, ygr)


def kernel(hidden_states, Wr, sg, su, sd, Wg, Wu, Wd):
    b, s, h = hidden_states.shape
    t = b * s
    x = hidden_states.reshape(t, h)
    e_num = Wr.shape[1]
    k = 2
    tk = t * k
    nt = tk // _M + e_num
    npad = nt * _M

    sgh = sg.astype(jnp.bfloat16)
    suh = su.astype(jnp.bfloat16)
    sdh = sd.astype(jnp.bfloat16)
    Wgh = Wg.astype(jnp.bfloat16)
    Wuh = Wu.astype(jnp.bfloat16)
    Wdh = Wd.astype(jnp.bfloat16)

    logits, ti, tw = _run_router(x, Wr)
    ti_flat = ti.reshape(tk)

    stok, te, pos = _run_dispatch(ti_flat, e_num, tk, nt)
    xs = _run_gather(stok, x, npad)
    y = _run_group_ffn(te, xs, Wgh, Wuh, Wdh, nt)
    shared = _run_shared(x, sgh, suh, sdh)
    yg = _run_ygather(pos, y)
    out = _run_combine(shared, tw, yg.reshape(t, 2 * h))
    return out.reshape(b, s, h), logits


# speedup vs baseline: 12.9350x; 1.2984x over previous
import functools

import jax
import jax.numpy as jnp
from jax import lax
from jax.experimental import pallas as pl
from jax.experimental.pallas import tpu as pltpu
from jax.experimental.pallas import tpu_sc as plsc

_BT = 256
_M = 128


def _silu(x):
    return x * jax.nn.sigmoid(x)


def _bf16_as_i32(a):
    n = a.shape[0]
    return lax.bitcast_convert_type(a.reshape(n, -1, 2), jnp.int32)


def _i32_as_bf16(a):
    n = a.shape[0]
    return lax.bitcast_convert_type(a, jnp.bfloat16).reshape(n, -1)


def _router_body(x_ref, wr_ref, logits_ref, ti_ref, tw_ref):
    xb = x_ref[...]
    logits = jnp.dot(xb, wr_ref[...], preferred_element_type=jnp.float32)
    logits_ref[...] = logits
    probs = jax.nn.softmax(logits, axis=-1)
    eidx = lax.broadcasted_iota(jnp.int32, probs.shape, 1)
    big = jnp.int32(probs.shape[1])
    m1 = jnp.max(probs, axis=-1, keepdims=True)
    i1 = jnp.min(jnp.where(probs == m1, eidx, big), axis=-1, keepdims=True)
    masked = jnp.where(eidx == i1, -jnp.inf, probs)
    m2 = jnp.max(masked, axis=-1, keepdims=True)
    i2 = jnp.min(jnp.where(masked == m2, eidx, big), axis=-1, keepdims=True)
    ssum = m1 + m2
    ti_ref[...] = jnp.concatenate([i1, i2], axis=1)
    tw_ref[...] = jnp.concatenate([m1 / ssum, m2 / ssum], axis=1)


def _run_router(x, Wr):
    t, h = x.shape
    e_num = Wr.shape[1]
    bt = 512
    return pl.pallas_call(
        _router_body,
        grid=(t // bt,),
        in_specs=[
            pl.BlockSpec((bt, h), lambda i: (i, 0)),
            pl.BlockSpec((h, e_num), lambda i: (0, 0)),
        ],
        out_specs=[
            pl.BlockSpec((bt, e_num), lambda i: (i, 0)),
            pl.BlockSpec((bt, 2), lambda i: (i, 0)),
            pl.BlockSpec((bt, 2), lambda i: (i, 0)),
        ],
        out_shape=[
            jax.ShapeDtypeStruct((t, e_num), jnp.float32),
            jax.ShapeDtypeStruct((t, 2), jnp.int32),
            jax.ShapeDtypeStruct((t, 2), jnp.float32),
        ],
        compiler_params=pltpu.CompilerParams(
            dimension_semantics=("parallel",),
        ),
    )(x, Wr)


def _shared_body(x_ref, sg_ref, su_ref, sd_ref, shared_ref):
    xh = x_ref[...].astype(jnp.bfloat16)
    g = jnp.dot(xh, sg_ref[...], preferred_element_type=jnp.float32)
    u = jnp.dot(xh, su_ref[...], preferred_element_type=jnp.float32)
    shared_ref[...] = jnp.dot((_silu(g) * u).astype(jnp.bfloat16), sd_ref[...],
                              preferred_element_type=jnp.float32)


def _run_shared(x, sgh, suh, sdh):
    t, h = x.shape
    fs = sgh.shape[1]
    return pl.pallas_call(
        _shared_body,
        grid=(t // _BT,),
        in_specs=[
            pl.BlockSpec((_BT, h), lambda i: (i, 0)),
            pl.BlockSpec((h, fs), lambda i: (0, 0)),
            pl.BlockSpec((h, fs), lambda i: (0, 0)),
            pl.BlockSpec((fs, h), lambda i: (0, 0)),
        ],
        out_specs=pl.BlockSpec((_BT, h), lambda i: (i, 0)),
        out_shape=jax.ShapeDtypeStruct((t, h), jnp.float32),
        compiler_params=pltpu.CompilerParams(
            dimension_semantics=("parallel",),
        ),
    )(x, sgh, suh, sdh)


def _dispatch_body(e_num, tk, nt, ti_ref, stok_ref, te_ref, pos_ref,
                   eid_v, stok_v, pos_v, merge_v, macc_v, zero_v, te_v,
                   pos_sh):
    cid = lax.axis_index("c")
    sid = lax.axis_index("s")
    nchunk = tk // 16
    iota16 = lax.iota(jnp.int32, 16)
    zeros16 = jnp.zeros((16,), jnp.int32)

    @pl.when((cid == 0) & (sid < e_num))
    def _expert_work():
        pltpu.sync_copy(ti_ref, eid_v)

        def zero_pos(i, _):
            pos_v[pl.ds(i * 16, 16)] = zeros16
            return 0
        lax.fori_loop(0, nchunk, zero_pos, 0)

        def zero_stok(i, _):
            stok_v[pl.ds(i * 16, 16)] = (i * 16 + iota16) & (tk // 2 - 1)
            return 0
        lax.fori_loop(0, nchunk + 8, zero_stok, 0)
        for zz in range(_M // 16):
            zero_v[pl.ds(zz * 16, 16)] = zz * 16 + iota16

        def cnt_body(i, acc):
            chunk = eid_v[pl.ds(i * 16, 16)]
            return tuple(acc[e] + (chunk == e).astype(jnp.int32)
                         for e in range(e_num))
        acc0 = tuple(zeros16 for _ in range(e_num))
        acc = lax.fori_loop(0, nchunk, cnt_body, acc0)
        cnt = [jnp.sum(a) for a in acc]
        ntile = [(c + (_M - 1)) // _M for c in cnt]
        base = []
        run = jnp.int32(0)
        for e in range(e_num):
            base.append(run)
            run = run + ntile[e] * _M
        used = run // _M
        my_base = jnp.int32(0)
        my_ntile = jnp.int32(0)
        for e in range(e_num):
            my_base = jnp.where(sid == e, base[e], my_base)
            my_ntile = jnp.where(sid == e, ntile[e], my_ntile)

        def scan_body(i, off):
            chunk = eid_v[pl.ds(i * 16, 16)]
            m = chunk == sid
            jvec = i * 16 + iota16
            tok = jvec >> 1
            cum = plsc.cumsum(m.astype(jnp.int32))
            plsc.store_compressed(stok_v.at[pl.ds(off, 16)], tok, mask=m)
            posvals = my_base + off + cum - 1
            plsc.store_scatter(pos_v, [jvec], posvals, mask=m)
            return off + jnp.max(cum)
        lax.fori_loop(0, nchunk, scan_body, jnp.int32(0))

        def dma_body(i, _):
            dst = pl.multiple_of(my_base + i * _M, _M)
            pltpu.sync_copy(stok_v.at[pl.ds(i * _M, _M)],
                            stok_ref.at[pl.ds(dst, _M)])
            return 0
        lax.fori_loop(0, my_ntile, dma_body, 0)

        pltpu.sync_copy(pos_v, pos_sh.at[sid])

        @pl.when(sid == 0)
        def _meta():
            for m_t in range(nt):
                @pl.when(m_t >= used)
                def _zt():
                    pltpu.sync_copy(zero_v, stok_ref.at[pl.ds(m_t * _M, _M)])
            tbs = [b // _M for b in base]
            for ch in range((nt + 15) // 16):
                mvec = ch * 16 + iota16
                te = jnp.zeros((16,), jnp.int32)
                for e in range(e_num):
                    inseg = (mvec >= tbs[e]) & (mvec < tbs[e] + ntile[e])
                    te = te + e * inseg.astype(jnp.int32)
                te_v[pl.ds(ch * 16, 16)] = te
            pltpu.sync_copy(te_v, te_ref)

    plsc.subcore_barrier()

    @pl.when(cid == 0)
    def _merge():
        seg = tk // 16
        for e in range(e_num):
            pltpu.sync_copy(pos_sh.at[e, pl.ds(sid * seg, seg)],
                            merge_v.at[e])

        def add_body(i, _):
            sl = pl.ds(i * 16, 16)
            s = merge_v[0, sl]
            for e in range(1, e_num):
                s = s + merge_v[e, sl]
            macc_v[sl] = s
            return 0
        lax.fori_loop(0, seg // 16, add_body, 0)
        dst = pl.multiple_of(sid * seg, seg)
        pltpu.sync_copy(macc_v, pos_ref.at[pl.ds(dst, seg)])


def _run_dispatch(ti_flat, e_num, tk, nt):
    mesh = plsc.VectorSubcoreMesh(core_axis_name="c", subcore_axis_name="s")
    seg = tk // 16
    body = functools.partial(_dispatch_body, e_num, tk, nt)
    f = pl.kernel(
        body,
        out_type=[
            jax.ShapeDtypeStruct((nt * _M,), jnp.int32),
            jax.ShapeDtypeStruct((((nt + 15) // 16) * 16,), jnp.int32),
            jax.ShapeDtypeStruct((tk,), jnp.int32),
        ],
        mesh=mesh,
        compiler_params=pltpu.CompilerParams(needs_layout_passes=False),
        scratch_types=[
            pltpu.VMEM((tk,), jnp.int32),
            pltpu.VMEM((tk + 128,), jnp.int32),
            pltpu.VMEM((tk,), jnp.int32),
            pltpu.VMEM((e_num, seg), jnp.int32),
            pltpu.VMEM((seg,), jnp.int32),
            pltpu.VMEM((_M,), jnp.int32),
            pltpu.VMEM((((nt + 15) // 16) * 16,), jnp.int32),
            pltpu.VMEM_SHARED((e_num, tk), jnp.int32),
        ],
    )
    return f(ti_flat)


def _gather_body(rows_per_w, h, src_ref, idx_hbm_ref, out_ref,
                 idx_v, buf0, buf1, sem0, sem1):
    wid = lax.axis_index("s") * 2 + lax.axis_index("c")
    s0 = pl.multiple_of(wid * rows_per_w, 8)
    ch = rows_per_w // 4
    pltpu.sync_copy(idx_hbm_ref.at[pl.ds(s0, rows_per_w)], idx_v)
    bufs = [buf0, buf1]
    sems = [sem0, sem1]
    descs = [None] * 4
    descs[0] = pltpu.async_copy(src_ref.at[idx_v.at[pl.ds(0, ch)]],
                                bufs[0], sems[0])
    for c in range(4):
        if c < 3:
            descs[c + 1] = pltpu.async_copy(
                src_ref.at[idx_v.at[pl.ds((c + 1) * ch, ch)]],
                bufs[(c + 1) % 2], sems[(c + 1) % 2])
        descs[c].wait()
        pltpu.sync_copy(bufs[c % 2],
                        out_ref.at[pl.ds(pl.multiple_of(s0 + c * ch, 8), ch)])


def _run_gather(stok, x, npad):
    t, h = x.shape
    rows_per_w = npad // 32
    mesh = plsc.VectorSubcoreMesh(core_axis_name="c", subcore_axis_name="s")
    body = functools.partial(_gather_body, rows_per_w, h)
    f = pl.kernel(
        body,
        out_type=jax.ShapeDtypeStruct((npad, h), jnp.float32),
        mesh=mesh,
        compiler_params=pltpu.CompilerParams(needs_layout_passes=False),
        scratch_types=[
            pltpu.VMEM((rows_per_w,), jnp.int32),
            pltpu.VMEM((rows_per_w // 4, h), jnp.float32),
            pltpu.VMEM((rows_per_w // 4, h), jnp.float32),
            pltpu.SemaphoreType.DMA,
            pltpu.SemaphoreType.DMA,
        ],
    )
    return f(x, stok)


def _group_ffn_body(te_ref, xs_ref, wg_ref, wu_ref, wd_ref, y_ref):
    xh = xs_ref[...].astype(jnp.bfloat16)
    g = jnp.dot(xh, wg_ref[0], preferred_element_type=jnp.float32)
    u = jnp.dot(xh, wu_ref[0], preferred_element_type=jnp.float32)
    y_ref[...] = jnp.dot((_silu(g) * u).astype(jnp.bfloat16), wd_ref[0],
                         preferred_element_type=jnp.float32)


def _run_group_ffn(te, xs, Wgh, Wuh, Wdh, nt):
    npad, h = xs.shape
    f = Wgh.shape[2]
    grid_spec = pltpu.PrefetchScalarGridSpec(
        num_scalar_prefetch=1,
        grid=(nt,),
        in_specs=[
            pl.BlockSpec((_M, h), lambda i, te_r: (i, 0)),
            pl.BlockSpec((1, h, f), lambda i, te_r: (te_r[i], 0, 0)),
            pl.BlockSpec((1, h, f), lambda i, te_r: (te_r[i], 0, 0)),
            pl.BlockSpec((1, f, h), lambda i, te_r: (te_r[i], 0, 0)),
        ],
        out_specs=pl.BlockSpec((_M, h), lambda i, te_r: (i, 0)),
    )
    return pl.pallas_call(
        _group_ffn_body,
        grid_spec=grid_spec,
        out_shape=jax.ShapeDtypeStruct((npad, h), jnp.float32),
        compiler_params=pltpu.CompilerParams(
            dimension_semantics=("arbitrary",),
        ),
    )(te, xs, Wgh, Wuh, Wdh)


def _run_ygather(pos, y):
    tk = pos.shape[0]
    h = y.shape[1]
    rows_per_w = tk // 32
    mesh = plsc.VectorSubcoreMesh(core_axis_name="c", subcore_axis_name="s")
    body = functools.partial(_gather_body, rows_per_w, h)
    f = pl.kernel(
        body,
        out_type=jax.ShapeDtypeStruct((tk, h), jnp.float32),
        mesh=mesh,
        compiler_params=pltpu.CompilerParams(needs_layout_passes=False),
        scratch_types=[
            pltpu.VMEM((rows_per_w,), jnp.int32),
            pltpu.VMEM((rows_per_w // 4, h), jnp.float32),
            pltpu.VMEM((rows_per_w // 4, h), jnp.float32),
            pltpu.SemaphoreType.DMA,
            pltpu.SemaphoreType.DMA,
        ],
    )
    return f(y, pos)


def _combine_body(sh_ref, w_ref, yg_ref, out_ref):
    h = sh_ref.shape[1]
    w0 = w_ref[:, 0:1]
    w1 = w_ref[:, 1:2]
    y0 = yg_ref[:, :h]
    y1 = yg_ref[:, h:]
    out_ref[...] = sh_ref[...] + w0 * y0 + w1 * y1


def _run_combine(shared, tw, ygr):
    t, h = shared.shape
    return pl.pallas_call(
        _combine_body,
        grid=(t // _BT,),
        in_specs=[
            pl.BlockSpec((_BT, h), lambda i: (i, 0)),
            pl.BlockSpec((_BT, 2), lambda i: (i, 0)),
            pl.BlockSpec((_BT, 2 * h), lambda i: (i, 0)),
        ],
        out_specs=pl.BlockSpec((_BT, h), lambda i: (i, 0)),
        out_shape=jax.ShapeDtypeStruct((t, h), jnp.float32),
        compiler_params=pltpu.CompilerParams(
            dimension_semantics=("parallel",),
        ),
    )(shared, tw, ygr)


def kernel(hidden_states, Wr, sg, su, sd, Wg, Wu, Wd):
    b, s, h = hidden_states.shape
    t = b * s
    x = hidden_states.reshape(t, h)
    e_num = Wr.shape[1]
    k = 2
    tk = t * k
    nt = tk // _M + e_num
    npad = nt * _M

    sgh = sg.astype(jnp.bfloat16)
    suh = su.astype(jnp.bfloat16)
    sdh = sd.astype(jnp.bfloat16)
    Wgh = Wg.astype(jnp.bfloat16)
    Wuh = Wu.astype(jnp.bfloat16)
    Wdh = Wd.astype(jnp.bfloat16)

    logits, ti, tw = _run_router(x, Wr)
    ti_flat = ti.reshape(tk)

    stok, te, pos = _run_dispatch(ti_flat, e_num, tk, nt)
    xs = _run_gather(stok, x, npad)
    y = _run_group_ffn(te, xs, Wgh, Wuh, Wdh, nt)
    shared = _run_shared(x, sgh, suh, sdh)
    yg = _run_ygather(pos, y)
    out = _run_combine(shared, tw, yg.reshape(t, 2 * h))
    return out.reshape(b, s, h), logits


# speedup vs baseline: 12.9670x; 1.0025x over previous
import functools

import jax
import jax.numpy as jnp
from jax import lax
from jax.experimental import pallas as pl
from jax.experimental.pallas import tpu as pltpu
from jax.experimental.pallas import tpu_sc as plsc

_BT = 256
_M = 128


def _silu(x):
    return x * jax.nn.sigmoid(x)


def _bf16_as_i32(a):
    n = a.shape[0]
    return lax.bitcast_convert_type(a.reshape(n, -1, 2), jnp.int32)


def _i32_as_bf16(a):
    n = a.shape[0]
    return lax.bitcast_convert_type(a, jnp.bfloat16).reshape(n, -1)


def _router_body(x_ref, wr_ref, logits_ref, ti_ref, tw_ref):
    xb = x_ref[...]
    logits = jnp.dot(xb, wr_ref[...], preferred_element_type=jnp.float32)
    logits_ref[...] = logits
    probs = jax.nn.softmax(logits, axis=-1)
    eidx = lax.broadcasted_iota(jnp.int32, probs.shape, 1)
    big = jnp.int32(probs.shape[1])
    m1 = jnp.max(probs, axis=-1, keepdims=True)
    i1 = jnp.min(jnp.where(probs == m1, eidx, big), axis=-1, keepdims=True)
    masked = jnp.where(eidx == i1, -jnp.inf, probs)
    m2 = jnp.max(masked, axis=-1, keepdims=True)
    i2 = jnp.min(jnp.where(masked == m2, eidx, big), axis=-1, keepdims=True)
    ssum = m1 + m2
    ti_ref[...] = jnp.concatenate([i1, i2], axis=1)
    tw_ref[...] = jnp.concatenate([m1 / ssum, m2 / ssum], axis=1)


def _run_router(x, Wr):
    t, h = x.shape
    e_num = Wr.shape[1]
    bt = 512
    return pl.pallas_call(
        _router_body,
        grid=(t // bt,),
        in_specs=[
            pl.BlockSpec((bt, h), lambda i: (i, 0)),
            pl.BlockSpec((h, e_num), lambda i: (0, 0)),
        ],
        out_specs=[
            pl.BlockSpec((bt, e_num), lambda i: (i, 0)),
            pl.BlockSpec((bt, 2), lambda i: (i, 0)),
            pl.BlockSpec((bt, 2), lambda i: (i, 0)),
        ],
        out_shape=[
            jax.ShapeDtypeStruct((t, e_num), jnp.float32),
            jax.ShapeDtypeStruct((t, 2), jnp.int32),
            jax.ShapeDtypeStruct((t, 2), jnp.float32),
        ],
        compiler_params=pltpu.CompilerParams(
            dimension_semantics=("parallel",),
        ),
    )(x, Wr)


def _shared_body(x_ref, sg_ref, su_ref, sd_ref, shared_ref):
    xh = x_ref[...].astype(jnp.bfloat16)
    g = jnp.dot(xh, sg_ref[...], preferred_element_type=jnp.float32)
    u = jnp.dot(xh, su_ref[...], preferred_element_type=jnp.float32)
    shared_ref[...] = jnp.dot((_silu(g) * u).astype(jnp.bfloat16), sd_ref[...],
                              preferred_element_type=jnp.float32)


def _run_shared(x, sgh, suh, sdh):
    t, h = x.shape
    fs = sgh.shape[1]
    return pl.pallas_call(
        _shared_body,
        grid=(t // _BT,),
        in_specs=[
            pl.BlockSpec((_BT, h), lambda i: (i, 0)),
            pl.BlockSpec((h, fs), lambda i: (0, 0)),
            pl.BlockSpec((h, fs), lambda i: (0, 0)),
            pl.BlockSpec((fs, h), lambda i: (0, 0)),
        ],
        out_specs=pl.BlockSpec((_BT, h), lambda i: (i, 0)),
        out_shape=jax.ShapeDtypeStruct((t, h), jnp.float32),
        compiler_params=pltpu.CompilerParams(
            dimension_semantics=("parallel",),
        ),
    )(x, sgh, suh, sdh)


def _dispatch_body(e_num, tk, nt, ti_ref, stok_ref, te_ref, pos_ref,
                   eid_v, stok_v, pos_v, merge_v, macc_v, zero_v, te_v,
                   pos_sh):
    cid = lax.axis_index("c")
    sid = lax.axis_index("s")
    nchunk = tk // 16
    iota16 = lax.iota(jnp.int32, 16)
    zeros16 = jnp.zeros((16,), jnp.int32)

    @pl.when((cid == 0) & (sid < e_num))
    def _expert_work():
        pltpu.sync_copy(ti_ref, eid_v)

        def zero_pos(i, _):
            pos_v[pl.ds(i * 16, 16)] = zeros16
            return 0
        lax.fori_loop(0, nchunk, zero_pos, 0)

        def zero_stok(i, _):
            stok_v[pl.ds(i * 16, 16)] = (i * 16 + iota16) & (tk // 2 - 1)
            return 0
        lax.fori_loop(0, nchunk + 8, zero_stok, 0)
        for zz in range(_M // 16):
            zero_v[pl.ds(zz * 16, 16)] = zz * 16 + iota16

        def cnt_body(i, acc):
            chunk = eid_v[pl.ds(i * 16, 16)]
            return tuple(acc[e] + (chunk == e).astype(jnp.int32)
                         for e in range(e_num))
        acc0 = tuple(zeros16 for _ in range(e_num))
        acc = lax.fori_loop(0, nchunk, cnt_body, acc0)
        cnt = [jnp.sum(a) for a in acc]
        ntile = [(c + (_M - 1)) // _M for c in cnt]
        base = []
        run = jnp.int32(0)
        for e in range(e_num):
            base.append(run)
            run = run + ntile[e] * _M
        used = run // _M
        my_base = jnp.int32(0)
        my_ntile = jnp.int32(0)
        for e in range(e_num):
            my_base = jnp.where(sid == e, base[e], my_base)
            my_ntile = jnp.where(sid == e, ntile[e], my_ntile)

        def scan_body(i, off):
            chunk = eid_v[pl.ds(i * 16, 16)]
            m = chunk == sid
            jvec = i * 16 + iota16
            tok = jvec >> 1
            cum = plsc.cumsum(m.astype(jnp.int32))
            plsc.store_compressed(stok_v.at[pl.ds(off, 16)], tok, mask=m)
            posvals = my_base + off + cum - 1
            plsc.store_scatter(pos_v, [jvec], posvals, mask=m)
            return off + jnp.max(cum)
        lax.fori_loop(0, nchunk, scan_body, jnp.int32(0))

        def dma_body(i, _):
            dst = pl.multiple_of(my_base + i * _M, _M)
            pltpu.sync_copy(stok_v.at[pl.ds(i * _M, _M)],
                            stok_ref.at[pl.ds(dst, _M)])
            return 0
        lax.fori_loop(0, my_ntile, dma_body, 0)

        pltpu.sync_copy(pos_v, pos_sh.at[sid])

        @pl.when(sid == 0)
        def _meta():
            for m_t in range(nt):
                @pl.when(m_t >= used)
                def _zt():
                    pltpu.sync_copy(zero_v, stok_ref.at[pl.ds(m_t * _M, _M)])
            tbs = [b // _M for b in base]
            for ch in range((nt + 15) // 16):
                mvec = ch * 16 + iota16
                te = jnp.zeros((16,), jnp.int32)
                for e in range(e_num):
                    inseg = (mvec >= tbs[e]) & (mvec < tbs[e] + ntile[e])
                    te = te + e * inseg.astype(jnp.int32)
                te = jnp.where(mvec < used, te, -1)
                te_v[pl.ds(ch * 16, 16)] = te
            pltpu.sync_copy(te_v, te_ref)

    plsc.subcore_barrier()

    @pl.when(cid == 0)
    def _merge():
        seg = tk // 16
        for e in range(e_num):
            pltpu.sync_copy(pos_sh.at[e, pl.ds(sid * seg, seg)],
                            merge_v.at[e])

        def add_body(i, _):
            sl = pl.ds(i * 16, 16)
            s = merge_v[0, sl]
            for e in range(1, e_num):
                s = s + merge_v[e, sl]
            macc_v[sl] = s
            return 0
        lax.fori_loop(0, seg // 16, add_body, 0)
        dst = pl.multiple_of(sid * seg, seg)
        pltpu.sync_copy(macc_v, pos_ref.at[pl.ds(dst, seg)])


def _run_dispatch(ti_flat, e_num, tk, nt):
    mesh = plsc.VectorSubcoreMesh(core_axis_name="c", subcore_axis_name="s")
    seg = tk // 16
    body = functools.partial(_dispatch_body, e_num, tk, nt)
    f = pl.kernel(
        body,
        out_type=[
            jax.ShapeDtypeStruct((nt * _M,), jnp.int32),
            jax.ShapeDtypeStruct((((nt + 15) // 16) * 16,), jnp.int32),
            jax.ShapeDtypeStruct((tk,), jnp.int32),
        ],
        mesh=mesh,
        compiler_params=pltpu.CompilerParams(needs_layout_passes=False),
        scratch_types=[
            pltpu.VMEM((tk,), jnp.int32),
            pltpu.VMEM((tk + 128,), jnp.int32),
            pltpu.VMEM((tk,), jnp.int32),
            pltpu.VMEM((e_num, seg), jnp.int32),
            pltpu.VMEM((seg,), jnp.int32),
            pltpu.VMEM((_M,), jnp.int32),
            pltpu.VMEM((((nt + 15) // 16) * 16,), jnp.int32),
            pltpu.VMEM_SHARED((e_num, tk), jnp.int32),
        ],
    )
    return f(ti_flat)


def _gather_body(rows_per_w, h, src_ref, idx_hbm_ref, out_ref,
                 idx_v, buf0, buf1, sem0, sem1):
    wid = lax.axis_index("s") * 2 + lax.axis_index("c")
    s0 = pl.multiple_of(wid * rows_per_w, 8)
    ch = rows_per_w // 4
    pltpu.sync_copy(idx_hbm_ref.at[pl.ds(s0, rows_per_w)], idx_v)
    bufs = [buf0, buf1]
    sems = [sem0, sem1]
    descs = [None] * 4
    descs[0] = pltpu.async_copy(src_ref.at[idx_v.at[pl.ds(0, ch)]],
                                bufs[0], sems[0])
    for c in range(4):
        if c < 3:
            descs[c + 1] = pltpu.async_copy(
                src_ref.at[idx_v.at[pl.ds((c + 1) * ch, ch)]],
                bufs[(c + 1) % 2], sems[(c + 1) % 2])
        descs[c].wait()
        pltpu.sync_copy(bufs[c % 2],
                        out_ref.at[pl.ds(pl.multiple_of(s0 + c * ch, 8), ch)])


def _run_gather(stok, x, npad):
    t, h = x.shape
    rows_per_w = npad // 32
    mesh = plsc.VectorSubcoreMesh(core_axis_name="c", subcore_axis_name="s")
    body = functools.partial(_gather_body, rows_per_w, h)
    f = pl.kernel(
        body,
        out_type=jax.ShapeDtypeStruct((npad, h), jnp.float32),
        mesh=mesh,
        compiler_params=pltpu.CompilerParams(needs_layout_passes=False),
        scratch_types=[
            pltpu.VMEM((rows_per_w,), jnp.int32),
            pltpu.VMEM((rows_per_w // 4, h), jnp.float32),
            pltpu.VMEM((rows_per_w // 4, h), jnp.float32),
            pltpu.SemaphoreType.DMA,
            pltpu.SemaphoreType.DMA,
        ],
    )
    return f(x, stok)


def _group_ffn_body(te_ref, xs_ref, wg_ref, wu_ref, wd_ref, y_ref):
    i = pl.program_id(0)

    @pl.when(te_ref[i] >= 0)
    def _compute():
        xh = xs_ref[...].astype(jnp.bfloat16)
        g = jnp.dot(xh, wg_ref[0], preferred_element_type=jnp.float32)
        u = jnp.dot(xh, wu_ref[0], preferred_element_type=jnp.float32)
        y_ref[...] = jnp.dot((_silu(g) * u).astype(jnp.bfloat16), wd_ref[0],
                             preferred_element_type=jnp.float32)


def _run_group_ffn(te, xs, Wgh, Wuh, Wdh, nt):
    npad, h = xs.shape
    f = Wgh.shape[2]
    grid_spec = pltpu.PrefetchScalarGridSpec(
        num_scalar_prefetch=1,
        grid=(nt,),
        in_specs=[
            pl.BlockSpec((_M, h), lambda i, te_r: (i, 0)),
            pl.BlockSpec((1, h, f),
                         lambda i, te_r: (jnp.maximum(te_r[i], 0), 0, 0)),
            pl.BlockSpec((1, h, f),
                         lambda i, te_r: (jnp.maximum(te_r[i], 0), 0, 0)),
            pl.BlockSpec((1, f, h),
                         lambda i, te_r: (jnp.maximum(te_r[i], 0), 0, 0)),
        ],
        out_specs=pl.BlockSpec((_M, h), lambda i, te_r: (i, 0)),
    )
    return pl.pallas_call(
        _group_ffn_body,
        grid_spec=grid_spec,
        out_shape=jax.ShapeDtypeStruct((npad, h), jnp.float32),
        compiler_params=pltpu.CompilerParams(
            dimension_semantics=("arbitrary",),
        ),
    )(te, xs, Wgh, Wuh, Wdh)


def _run_ygather(pos, y):
    tk = pos.shape[0]
    h = y.shape[1]
    rows_per_w = tk // 32
    mesh = plsc.VectorSubcoreMesh(core_axis_name="c", subcore_axis_name="s")
    body = functools.partial(_gather_body, rows_per_w, h)
    f = pl.kernel(
        body,
        out_type=jax.ShapeDtypeStruct((tk, h), jnp.float32),
        mesh=mesh,
        compiler_params=pltpu.CompilerParams(needs_layout_passes=False),
        scratch_types=[
            pltpu.VMEM((rows_per_w,), jnp.int32),
            pltpu.VMEM((rows_per_w // 4, h), jnp.float32),
            pltpu.VMEM((rows_per_w // 4, h), jnp.float32),
            pltpu.SemaphoreType.DMA,
            pltpu.SemaphoreType.DMA,
        ],
    )
    return f(y, pos)


def _combine_body(sh_ref, w_ref, yg_ref, out_ref):
    h = sh_ref.shape[1]
    w0 = w_ref[:, 0:1]
    w1 = w_ref[:, 1:2]
    y0 = yg_ref[:, :h]
    y1 = yg_ref[:, h:]
    out_ref[...] = sh_ref[...] + w0 * y0 + w1 * y1


def _run_combine(shared, tw, ygr):
    t, h = shared.shape
    return pl.pallas_call(
        _combine_body,
        grid=(t // _BT,),
        in_specs=[
            pl.BlockSpec((_BT, h), lambda i: (i, 0)),
            pl.BlockSpec((_BT, 2), lambda i: (i, 0)),
            pl.BlockSpec((_BT, 2 * h), lambda i: (i, 0)),
        ],
        out_specs=pl.BlockSpec((_BT, h), lambda i: (i, 0)),
        out_shape=jax.ShapeDtypeStruct((t, h), jnp.float32),
        compiler_params=pltpu.CompilerParams(
            dimension_semantics=("parallel",),
        ),
    )(shared, tw, ygr)


def kernel(hidden_states, Wr, sg, su, sd, Wg, Wu, Wd):
    b, s, h = hidden_states.shape
    t = b * s
    x = hidden_states.reshape(t, h)
    e_num = Wr.shape[1]
    k = 2
    tk = t * k
    nt = tk // _M + e_num
    npad = nt * _M

    sgh = sg.astype(jnp.bfloat16)
    suh = su.astype(jnp.bfloat16)
    sdh = sd.astype(jnp.bfloat16)
    Wgh = Wg.astype(jnp.bfloat16)
    Wuh = Wu.astype(jnp.bfloat16)
    Wdh = Wd.astype(jnp.bfloat16)

    logits, ti, tw = _run_router(x, Wr)
    ti_flat = ti.reshape(tk)

    stok, te, pos = _run_dispatch(ti_flat, e_num, tk, nt)
    xs = _run_gather(stok, x, npad)
    shared = _run_shared(x, sgh, suh, sdh)
    y = _run_group_ffn(te, xs, Wgh, Wuh, Wdh, nt)
    yg = _run_ygather(pos, y)
    out = _run_combine(shared, tw, yg.reshape(t, 2 * h))
    return out.reshape(b, s, h), logits
